# Initial kernel scaffold; baseline (speedup 1.0000x reference)
import jax
import jax.numpy as jnp
from jax.experimental import pallas as pl


def kernel(x_row, x_col, edge_index, edge_weight, W1_c2r, b1_c2r, W2_c2r, b2_c2r, W1_r2c, b1_r2c, W2_r2c, b2_r2c, eps_c2r, eps_r2c):
    raise NotImplementedError("write your pallas kernel here")


# speedup vs baseline: 3.7618x; 3.7618x over previous
import functools

import jax
import jax.numpy as jnp
from jax import lax
from jax.experimental import pallas as pl
from jax.experimental.pallas import tpu as pltpu
from jax.experimental.pallas import tpu_sc as plsc

N_ROW = 10000
N_COL = 10000
E = 320000
D = 128

NC = 2
NS = 16
NW = NC * NS
EPT = E // NW
CH = 80
NCHUNK = EPT // CH
NPAD = 10112
RPT = NPAD // NS


def _sc_agg_body(table_hbm, gidx_hbm, sidx_hbm, ew_hbm, zeros_hbm, out_hbm,
                 gi_v, si_v, ew_v, rows_v, acc, sem):
    c = lax.axis_index("c")
    s = lax.axis_index("s")
    wid = s * NC + c

    pltpu.sync_copy(zeros_hbm.at[pl.ds(s * RPT, RPT)], acc.at[pl.ds(s * RPT, RPT)])
    plsc.subcore_barrier()

    ebase = wid * EPT

    def chunk(k, carry):
        off = ebase + k * CH
        pltpu.sync_copy(gidx_hbm.at[pl.ds(off, CH)], gi_v)
        pltpu.sync_copy(sidx_hbm.at[pl.ds(off, CH)], si_v)
        pltpu.sync_copy(ew_hbm.at[pl.ds(off, CH)], ew_v)
        pltpu.async_copy(table_hbm.at[gi_v], rows_v, sem).wait()

        def grp(g, carry2):
            base = g * 16
            ew16 = ew_v[pl.ds(base, 16)]
            for j in range(16):
                e = base + j
                w = lax.gather(
                    ew16, jnp.full((16, 1), j, jnp.int32),
                    lax.GatherDimensionNumbers(
                        offset_dims=(), collapsed_slice_dims=(0,),
                        start_index_map=(0,)),
                    (1,), mode=lax.GatherScatterMode.PROMISE_IN_BOUNDS)
                for d in range(8):
                    v = rows_v[e, pl.ds(d * 16, 16)]
                    rows_v[e, pl.ds(d * 16, 16)] = v * w
            return carry2

        lax.fori_loop(0, CH // 16, grp, 0)
        pltpu.sync_copy(rows_v, acc.at[si_v], add=True)
        return carry

    lax.fori_loop(0, NCHUNK, chunk, 0)

    plsc.subcore_barrier()
    pltpu.sync_copy(acc.at[pl.ds(s * RPT, RPT)], out_hbm.at[c, pl.ds(s * RPT, RPT)])


_sc_agg = functools.partial(
    pl.kernel,
    out_type=jax.ShapeDtypeStruct((NC, NPAD, D), jnp.float32),
    mesh=plsc.VectorSubcoreMesh(core_axis_name="c", subcore_axis_name="s"),
    scratch_types=[
        pltpu.VMEM((CH,), jnp.int32),
        pltpu.VMEM((CH,), jnp.int32),
        pltpu.VMEM((CH,), jnp.float32),
        pltpu.VMEM((CH, D), jnp.float32),
        pltpu.VMEM_SHARED((NPAD, D), jnp.float32),
        pltpu.SemaphoreType.DMA,
    ],
)(_sc_agg_body)


def _mlp_block(x_ref, p0_ref, p1_ref, w1_ref, b1_ref, w2_ref, b2_ref,
               eps_ref, pre_ref, out_ref):
    x = x_ref[...]
    a = eps_ref[0, 0] * x + p0_ref[...] + p1_ref[...]
    h = jnp.dot(a, w1_ref[...], preferred_element_type=jnp.float32) + b1_ref[...]
    h = jnp.where(h >= 0, h, 0.01 * h)
    h = jnp.dot(h, w2_ref[...], preferred_element_type=jnp.float32) + b2_ref[...]
    h = jnp.where(h >= 0, h, 0.01 * h)
    pre_ref[...] = h
    out_ref[...] = h + x


def _tc_mlp(x, p0, p1, w1t, b1, w2t, b2, eps1):
    n = x.shape[0]
    br = 1000
    row_spec = pl.BlockSpec((br, D), lambda i: (i, 0))
    full_spec = pl.BlockSpec((D, D), lambda i: (0, 0))
    bias_spec = pl.BlockSpec((1, D), lambda i: (0, 0))
    return pl.pallas_call(
        _mlp_block,
        grid=(n // br,),
        in_specs=[row_spec, row_spec, row_spec, full_spec, bias_spec,
                  full_spec, bias_spec,
                  pl.BlockSpec(memory_space=pltpu.SMEM)],
        out_specs=[row_spec, row_spec],
        out_shape=[jax.ShapeDtypeStruct((n, D), jnp.float32)] * 2,
    )(x, p0, p1, w1t, b1, w2t, b2, eps1)


def kernel(x_row, x_col, edge_index, edge_weight,
           W1_c2r, b1_c2r, W2_c2r, b2_c2r,
           W1_r2c, b1_r2c, W2_r2c, b2_r2c,
           eps_c2r, eps_r2c):
    src = edge_index[0].astype(jnp.int32)
    dst = edge_index[1].astype(jnp.int32)
    ew = edge_weight.astype(jnp.float32)
    zeros = jnp.zeros((NPAD, D), jnp.float32)

    pr = _sc_agg(x_col, src, dst, ew, zeros)
    h_pre, h_row = _tc_mlp(
        x_row, pr[0, :N_ROW], pr[1, :N_ROW],
        W1_c2r.T, b1_c2r[None, :], W2_c2r.T, b2_c2r[None, :],
        jnp.reshape(1.0 + eps_c2r, (1, 1)))

    pc = _sc_agg(h_pre, dst, src, ew, zeros)
    _, h_col = _tc_mlp(
        x_col, pc[0, :N_COL], pc[1, :N_COL],
        W1_r2c.T, b1_r2c[None, :], W2_r2c.T, b2_r2c[None, :],
        jnp.reshape(1.0 + eps_r2c, (1, 1)))

    return (h_row, h_col)
